# baseline (speedup 1.0000x reference)
import jax
import jax.numpy as jnp
from jax.experimental import pallas as pl
from jax.experimental.pallas import tpu as pltpu

_B = 1024
_D = 128
_K = 1024
_EPS = 1e-6
_NB = 8
_BB = _B // _NB


def _som_body(x_ref, w_ref, loc_ref, locs_ref, loss_ref, wsq_ref):
    i = pl.program_id(0)

    @pl.when(i == 0)
    def _init():
        w = w_ref[...]
        wsq_ref[...] = jnp.sum(w * w, axis=0, keepdims=True)
        loss_ref[...] = jnp.zeros((1, 1), jnp.float32)

    x = x_ref[...] + _EPS
    xsq = jnp.sum(x * x, axis=1, keepdims=True)
    cross = jax.lax.dot_general(
        x, w_ref[...], (((1,), (0,)), ((), ())),
        precision=jax.lax.Precision.HIGHEST,
        preferred_element_type=jnp.float32,
    )
    d2 = jnp.maximum(xsq - 2.0 * cross + wsq_ref[...], 0.0)
    dists = jnp.sqrt(d2)
    mins = jnp.min(dists, axis=1, keepdims=True)
    kiota = jax.lax.broadcasted_iota(jnp.int32, (_BB, _K), 1)
    idx = jnp.min(jnp.where(dists == mins, kiota, _K), axis=1, keepdims=True)
    onehot = (kiota == idx).astype(jnp.float32)
    locs_ref[...] = jax.lax.dot_general(
        onehot, loc_ref[...], (((1,), (0,)), ((), ())),
        preferred_element_type=jnp.float32,
    )
    loss_ref[...] += jnp.sum(mins, axis=0, keepdims=True) / _B


def kernel(input, weight, locations):
    locs, loss = pl.pallas_call(
        _som_body,
        grid=(_NB,),
        in_specs=[
            pl.BlockSpec((_BB, _D), lambda i: (i, 0)),
            pl.BlockSpec((_D, _K), lambda i: (0, 0)),
            pl.BlockSpec((_K, 2), lambda i: (0, 0)),
        ],
        out_specs=(
            pl.BlockSpec((_BB, 2), lambda i: (i, 0)),
            pl.BlockSpec((1, 1), lambda i: (0, 0)),
        ),
        out_shape=(
            jax.ShapeDtypeStruct((_B, 2), jnp.float32),
            jax.ShapeDtypeStruct((1, 1), jnp.float32),
        ),
        scratch_shapes=[pltpu.VMEM((1, _K), jnp.float32)],
    )(input, weight, locations)
    return locs.reshape(_B, 1, 2), loss[0, 0]


# speedup vs baseline: 1.2598x; 1.2598x over previous
import jax
import jax.numpy as jnp
from jax.experimental import pallas as pl
from jax.experimental.pallas import tpu as pltpu

_B = 1024
_D = 128
_K = 1024
_EPS = 1e-6
_NC = 4
_KC = _K // _NC


def _som_body(x_ref, w_ref, loc_ref, locs_ref, loss_ref):
    x = x_ref[...] + _EPS
    xsq = jnp.sum(x * x, axis=1, keepdims=True)
    w = w_ref[...]
    halfwsq = 0.5 * jnp.sum(w * w, axis=0, keepdims=True)
    maxv = None
    idxv = None
    for c in range(_NC):
        wc = w[:, c * _KC:(c + 1) * _KC]
        cross = jax.lax.dot_general(
            x, wc, (((1,), (0,)), ((), ())),
            precision=jax.lax.Precision.HIGHEST,
            preferred_element_type=jnp.float32,
        )
        r = cross - halfwsq[:, c * _KC:(c + 1) * _KC]
        m_c = jnp.max(r, axis=1, keepdims=True)
        kio = jax.lax.broadcasted_iota(jnp.int32, (_B, _KC), 1) + c * _KC
        i_c = jnp.min(jnp.where(r == m_c, kio, _K), axis=1, keepdims=True)
        if c == 0:
            maxv, idxv = m_c, i_c
        else:
            idxv = jnp.where(m_c > maxv, i_c, idxv)
            maxv = jnp.maximum(maxv, m_c)
    kiota = jax.lax.broadcasted_iota(jnp.int32, (_B, _K), 1)
    onehot = (kiota == idxv).astype(jnp.bfloat16)
    locs_ref[...] = jax.lax.dot_general(
        onehot, loc_ref[...].astype(jnp.bfloat16), (((1,), (0,)), ((), ())),
        preferred_element_type=jnp.float32,
    )
    mind = jnp.sqrt(jnp.maximum(xsq - 2.0 * maxv, 0.0))
    loss_ref[...] = jnp.sum(mind, axis=0, keepdims=True) / _B


def kernel(input, weight, locations):
    locs, loss = pl.pallas_call(
        _som_body,
        out_shape=(
            jax.ShapeDtypeStruct((_B, 2), jnp.float32),
            jax.ShapeDtypeStruct((1, 1), jnp.float32),
        ),
    )(input, weight, locations)
    return locs.reshape(_B, 1, 2), loss[0, 0]


# speedup vs baseline: 1.2825x; 1.0180x over previous
import jax
import jax.numpy as jnp
from jax.experimental import pallas as pl
from jax.experimental.pallas import tpu as pltpu

_B = 1024
_D = 128
_K = 1024
_EPS = 1e-6
_HB = _B // 2
_HK = _K // 2


def _rank_chunk(x, w, halfwsq, koff):
    cross = jax.lax.dot_general(
        x, w, (((1,), (0,)), ((), ())),
        precision=jax.lax.Precision.HIGHEST,
        preferred_element_type=jnp.float32,
    )
    r = cross - halfwsq
    m = jnp.max(r, axis=1, keepdims=True)
    kio = jax.lax.broadcasted_iota(jnp.int32, (_HB, _HK), 1) + koff
    i = jnp.min(jnp.where(r == m, kio, _K), axis=1, keepdims=True)
    return m, i


def _som_body(x_hbm, w_hbm, loc_hbm, locs_ref, loss_ref,
              x_v, w_v, loc_v, sems):
    cx0 = pltpu.make_async_copy(x_hbm.at[pl.ds(0, _HB)], x_v.at[pl.ds(0, _HB)], sems.at[0])
    cw0 = pltpu.make_async_copy(w_hbm.at[:, pl.ds(0, _HK)], w_v.at[:, pl.ds(0, _HK)], sems.at[1])
    cx1 = pltpu.make_async_copy(x_hbm.at[pl.ds(_HB, _HB)], x_v.at[pl.ds(_HB, _HB)], sems.at[2])
    cw1 = pltpu.make_async_copy(w_hbm.at[:, pl.ds(_HK, _HK)], w_v.at[:, pl.ds(_HK, _HK)], sems.at[3])
    cloc = pltpu.make_async_copy(loc_hbm, loc_v, sems.at[4])
    cx0.start()
    cw0.start()
    cx1.start()
    cw1.start()
    cloc.start()

    cx0.wait()
    cw0.wait()
    x0 = x_v[pl.ds(0, _HB), :] + _EPS
    xsq0 = jnp.sum(x0 * x0, axis=1, keepdims=True)
    w0 = w_v[:, pl.ds(0, _HK)]
    hw0 = 0.5 * jnp.sum(w0 * w0, axis=0, keepdims=True)
    m00, i00 = _rank_chunk(x0, w0, hw0, 0)

    cx1.wait()
    x1 = x_v[pl.ds(_HB, _HB), :] + _EPS
    xsq1 = jnp.sum(x1 * x1, axis=1, keepdims=True)
    m10, i10 = _rank_chunk(x1, w0, hw0, 0)

    cw1.wait()
    w1 = w_v[:, pl.ds(_HK, _HK)]
    hw1 = 0.5 * jnp.sum(w1 * w1, axis=0, keepdims=True)
    m01, i01 = _rank_chunk(x0, w1, hw1, _HK)
    m11, i11 = _rank_chunk(x1, w1, hw1, _HK)

    idx0 = jnp.where(m01 > m00, i01, i00)
    maxv0 = jnp.maximum(m00, m01)
    idx1 = jnp.where(m11 > m10, i11, i10)
    maxv1 = jnp.maximum(m10, m11)

    cloc.wait()
    locb = loc_v[...].astype(jnp.bfloat16)
    kiota = jax.lax.broadcasted_iota(jnp.int32, (_HB, _K), 1)
    oh0 = (kiota == idx0).astype(jnp.bfloat16)
    oh1 = (kiota == idx1).astype(jnp.bfloat16)
    locs_ref[pl.ds(0, _HB), :] = jax.lax.dot_general(
        oh0, locb, (((1,), (0,)), ((), ())), preferred_element_type=jnp.float32)
    locs_ref[pl.ds(_HB, _HB), :] = jax.lax.dot_general(
        oh1, locb, (((1,), (0,)), ((), ())), preferred_element_type=jnp.float32)

    mind0 = jnp.sqrt(jnp.maximum(xsq0 - 2.0 * maxv0, 0.0))
    mind1 = jnp.sqrt(jnp.maximum(xsq1 - 2.0 * maxv1, 0.0))
    loss_ref[...] = (jnp.sum(mind0, axis=0, keepdims=True)
                     + jnp.sum(mind1, axis=0, keepdims=True)) / _B


def kernel(input, weight, locations):
    locs, loss = pl.pallas_call(
        _som_body,
        in_specs=[
            pl.BlockSpec(memory_space=pl.ANY),
            pl.BlockSpec(memory_space=pl.ANY),
            pl.BlockSpec(memory_space=pl.ANY),
        ],
        out_shape=(
            jax.ShapeDtypeStruct((_B, 2), jnp.float32),
            jax.ShapeDtypeStruct((1, 1), jnp.float32),
        ),
        scratch_shapes=[
            pltpu.VMEM((_B, _D), jnp.float32),
            pltpu.VMEM((_D, _K), jnp.float32),
            pltpu.VMEM((_K, 2), jnp.float32),
            pltpu.SemaphoreType.DMA((8,)),
        ],
    )(input, weight, locations)
    return locs.reshape(_B, 1, 2), loss[0, 0]
